# baseline (speedup 1.0000x reference)
import functools

import jax
import jax.numpy as jnp
from jax import lax
from jax.experimental import pallas as pl
from jax.experimental.pallas import tpu as pltpu
from jax.experimental.pallas import tpu_sc as plsc

N = 10000
K = 2
C = 128
H = 8
D = 16
ES = 160000
EC = 20000
TN = K * N

HG = H // 2
CG = HG * D
E_SC = ES + EC
TILES = 16
CHUNK = 128
NCHUNK = 88
EPT = NCHUNK * CHUNK
EPAD = TILES * EPT
ACC_ROWS = 10112
RPT = ACC_ROWS // TILES
ACC_W = CG + 16

ROWS_BLK = 800
GRID = TN // ROWS_BLK


def _silu(z):
    return z * jax.nn.sigmoid(z)


def _tc1_body(x_ref, c7_ref, te_ref, msel_ref,
              sW1_ref, sb1_ref, sW2_ref, sb2_ref,
              mW1_ref, mb1_ref, mW2_ref, mb2_ref,
              Wq_ref, bq_ref, Wk_ref, bk_ref, Wv_ref, bv_ref,
              y_ref, q0_ref, q1_ref, kv0_ref, kv1_ref):
    x = x_ref[...]
    ctx = jnp.sin(30.0 * (c7_ref[...] @ sW1_ref[...] + sb1_ref[...]))
    ctx = ctx @ sW2_ref[...] + sb2_ref[...]
    hmod = _silu(te_ref[...] @ mW1_ref[...] + mb1_ref[...])
    hmod = hmod @ mW2_ref[...] + mb2_ref[...]
    a = hmod[:, :C]
    bmod = hmod[:, C:2 * C]
    z = x + ctx
    mu = jnp.mean(z, axis=-1, keepdims=True)
    zc = z - mu
    var = jnp.mean(zc * zc, axis=-1, keepdims=True)
    y = (a + 1.0) * (zc * lax.rsqrt(var + 1e-5)) + bmod
    msel = msel_ref[...]
    q = y @ Wq_ref[...] + bq_ref[...]
    qn = q / (jnp.sqrt((q * q) @ msel) + 1e-6)
    k = y @ Wk_ref[...] + bk_ref[...]
    kn = k / (jnp.sqrt((k * k) @ msel) + 1e-6)
    v = y @ Wv_ref[...] + bv_ref[...]
    y_ref[...] = y
    q0_ref[...] = qn[:, :CG]
    q1_ref[...] = qn[:, CG:]
    kv0_ref[:, :CG] = kn[:, :CG]
    kv0_ref[:, CG:] = v[:, :CG]
    kv1_ref[:, :CG] = kn[:, CG:]
    kv1_ref[:, CG:] = v[:, CG:]


def _tc_pre(x2, c7, te, msel, sW1, sb1, sW2, sb2, mW1, mb1, mW2, mb2,
            Wq, bq, Wk, bk, Wv, bv):
    full = lambda s: pl.BlockSpec(s, lambda i: (0, 0))
    row = lambda w: pl.BlockSpec((ROWS_BLK, w), lambda i: (i, 0))
    return pl.pallas_call(
        _tc1_body,
        grid=(GRID,),
        in_specs=[row(C), row(7), full((1, 256)), full((C, C)),
                  full((7, C)), full((1, C)), full((C, C)), full((1, C)),
                  full((256, 256)), full((1, 256)), full((256, 3 * C)), full((1, 3 * C)),
                  full((C, C)), full((1, C)), full((C, C)), full((1, C)),
                  full((C, C)), full((1, C))],
        out_specs=[row(C), row(CG), row(CG), row(2 * CG), row(2 * CG)],
        out_shape=[jax.ShapeDtypeStruct((TN, C), jnp.float32),
                   jax.ShapeDtypeStruct((TN, CG), jnp.float32),
                   jax.ShapeDtypeStruct((TN, CG), jnp.float32),
                   jax.ShapeDtypeStruct((TN, 2 * CG), jnp.float32),
                   jax.ShapeDtypeStruct((TN, 2 * CG), jnp.float32)],
    )(x2, c7, te, msel, sW1, sb1, sW2, sb2, mW1, mb1, mW2, mb2,
      Wq, bq, Wk, bk, Wv, bv)


def _sc_edges(q0, q1, kv0, kv1, eidx, zrow):
    mesh = plsc.VectorSubcoreMesh(core_axis_name="c", subcore_axis_name="s")

    @functools.partial(
        pl.kernel,
        mesh=mesh,
        compiler_params=pltpu.CompilerParams(
            needs_layout_passes=False, use_tc_tiling_on_sc=False),
        out_type=[jax.ShapeDtypeStruct((K, ACC_ROWS, CG), jnp.float32),
                  jax.ShapeDtypeStruct((K, ACC_ROWS, CG), jnp.float32),
                  jax.ShapeDtypeStruct((K, ACC_ROWS, 16), jnp.float32),
                  jax.ShapeDtypeStruct((K, ACC_ROWS, 16), jnp.float32)],
        scratch_types=[
            pltpu.VMEM((4, 3, CHUNK), jnp.int32),
            pltpu.VMEM((2, CHUNK), jnp.int32),
            pltpu.VMEM((2, CHUNK, CG), jnp.float32),
            pltpu.VMEM((2, CHUNK, 2 * CG), jnp.float32),
            pltpu.VMEM((2, CHUNK, ACC_W), jnp.float32),
            pltpu.VMEM_SHARED((ACC_ROWS, ACC_W), jnp.float32),
            pltpu.SemaphoreType.DMA,
            pltpu.SemaphoreType.DMA,
            pltpu.SemaphoreType.DMA,
            pltpu.SemaphoreType.DMA,
            pltpu.SemaphoreType.DMA,
            pltpu.SemaphoreType.DMA,
            pltpu.SemaphoreType.DMA,
            pltpu.SemaphoreType.DMA,
        ],
    )
    def _k(q0_hbm, q1_hbm, kv0_hbm, kv1_hbm, eidx_hbm, zr_hbm,
           onum0_hbm, onum1_hbm, oden0_hbm, oden1_hbm,
           ibuf, sidx, qv, kvv, pbuf, acc,
           si0, si1, si2, si3, sg0, sg1, ss0, ss1):
        sem_i = (si0, si1, si2, si3)
        sem_g = (sg0, sg1)
        sem_s = (ss0, ss1)
        c = lax.axis_index("c")
        s = lax.axis_index("s")
        rows = pl.ds(s * RPT, RPT)
        pltpu.sync_copy(zr_hbm.at[pl.ds(0, CHUNK)], pbuf.at[0])
        pltpu.sync_copy(zr_hbm.at[pl.ds(0, CHUNK)], pbuf.at[1])
        lane = lax.iota(jnp.int32, 16)

        def idx_start(i, slot, c=c, s=s):
            pltpu.async_copy(eidx_hbm.at[c, s, i], ibuf.at[slot], sem_i[slot])

        def idx_wait(i, slot, c=c, s=s):
            pltpu.make_async_copy(
                eidx_hbm.at[c, s, i], ibuf.at[slot], sem_i[slot]).wait()

        for hg, (q_hbm, kv_hbm, onum_hbm, oden_hbm) in enumerate(
                [(q0_hbm, kv0_hbm, onum0_hbm, oden0_hbm),
                 (q1_hbm, kv1_hbm, onum1_hbm, oden1_hbm)]):
            pltpu.sync_copy(zr_hbm, acc.at[rows])
            plsc.subcore_barrier()

            def gathers_start(i, slot, g2, q_hbm=q_hbm, kv_hbm=kv_hbm):
                pltpu.async_copy(q_hbm.at[ibuf.at[slot, 0]], qv.at[g2],
                                 sem_g[g2])
                pltpu.async_copy(kv_hbm.at[ibuf.at[slot, 1]], kvv.at[g2],
                                 sem_g[g2])

            def gathers_wait(i, slot, g2, q_hbm=q_hbm, kv_hbm=kv_hbm):
                pltpu.make_async_copy(
                    q_hbm.at[ibuf.at[slot, 0]], qv.at[g2], sem_g[g2]).wait()
                pltpu.make_async_copy(
                    kv_hbm.at[ibuf.at[slot, 1]], kvv.at[g2], sem_g[g2]).wait()

            idx_start(0, 0)
            idx_start(1, 1)
            idx_wait(0, 0)
            gathers_start(0, 0, 0)
            idx_start(2, 2)

            def quad_body(ii, carry, hg=hg):
                for b in (0, 1, 2, 3):
                    i = 4 * ii + b
                    g2 = b % 2
                    gathers_wait(i, b, g2)

                    @pl.when(4 * ii + b >= 2)
                    def _():
                        pltpu.make_async_copy(
                            pbuf.at[g2], acc.at[sidx.at[g2]], sem_s[g2]).wait()

                    @pl.when(i + 3 < NCHUNK)
                    def _():
                        idx_start(i + 3, (b + 3) % 4)

                    @pl.when(i + 1 < NCHUNK)
                    def _():
                        idx_wait(i + 1, (b + 1) % 4)
                        gathers_start(i + 1, (b + 1) % 4, (g2 + 1) % 2)

                    def edge_body(e, carry2, g2=g2):
                        pvec = jnp.zeros((16,), jnp.float32)
                        for h in range(HG):
                            qh = qv[g2, e, pl.ds(h * D, D)]
                            kh = kvv[g2, e, pl.ds(h * D, D)]
                            dot = jnp.sum(qh * kh)
                            pexp = jnp.exp(jnp.full((16,), dot * 0.25,
                                                    jnp.float32))
                            pvec = jnp.where(lane == h, pexp, pvec)
                            vh = kvv[g2, e, pl.ds(CG + h * D, D)]
                            pbuf[g2, e, pl.ds(h * D, D)] = pexp * vh
                        pbuf[g2, e, pl.ds(CG, 16)] = pvec
                        return carry2

                    lax.fori_loop(0, CHUNK, edge_body, 0)
                    for t in range(CHUNK // 16):
                        sidx[g2, pl.ds(t * 16, 16)] = ibuf[b, 2, pl.ds(t * 16, 16)]
                    pltpu.async_copy(pbuf.at[g2], acc.at[sidx.at[g2]],
                                     sem_s[g2], add=True)
                return carry

            lax.fori_loop(0, NCHUNK // 4, quad_body, 0)
            for g2 in (0, 1):
                pltpu.make_async_copy(
                    pbuf.at[g2], acc.at[sidx.at[g2]], sem_s[g2]).wait()
            plsc.subcore_barrier()
            pltpu.sync_copy(acc.at[rows, pl.ds(0, CG)], onum_hbm.at[c, rows])
            pltpu.sync_copy(acc.at[rows, pl.ds(CG, 16)], oden_hbm.at[c, rows])

    return _k(q0, q1, kv0, kv1, eidx, zrow)


def _tc2_body(x_ref, y_ref, num0_ref, num1_ref, den_ref, dsel_ref, te_ref,
              mW1_ref, mb1_ref, mW2_ref, mb2_ref,
              Wo_ref, bo_ref, fW1_ref, fb1_ref, fW2_ref, fb2_ref,
              o_ref):
    hmod = _silu(te_ref[...] @ mW1_ref[...] + mb1_ref[...])
    hmod = hmod @ mW2_ref[...] + mb2_ref[...]
    cmod = hmod[:, 2 * C:]
    rden = 1.0 / (den_ref[...] @ dsel_ref[...] + 1e-9)
    attn = ((num0_ref[...] * rden[:, :CG]) @ Wo_ref[:CG, :]
            + (num1_ref[...] * rden[:, CG:]) @ Wo_ref[CG:, :] + bo_ref[...])
    y2 = y_ref[...] + attn
    f = _silu(y2 @ fW1_ref[...] + fb1_ref[...]) @ fW2_ref[...] + fb2_ref[...]
    o_ref[...] = (x_ref[...] + cmod * f) * lax.rsqrt(1.0 + cmod * cmod)


def _tc_post(x2, y, num0, num1, den, dsel, te, mW1, mb1, mW2, mb2,
             Wo, bo, fW1, fb1, fW2, fb2):
    full = lambda s: pl.BlockSpec(s, lambda i: (0, 0))
    row = lambda w: pl.BlockSpec((ROWS_BLK, w), lambda i: (i, 0))
    return pl.pallas_call(
        _tc2_body,
        grid=(GRID,),
        in_specs=[row(C), row(C), row(CG), row(CG), row(H), full((H, C)),
                  full((1, 256)),
                  full((256, 256)), full((1, 256)), full((256, 3 * C)), full((1, 3 * C)),
                  full((C, C)), full((1, C)),
                  full((C, 4 * C)), full((1, 4 * C)), full((4 * C, C)), full((1, C))],
        out_specs=row(C),
        out_shape=jax.ShapeDtypeStruct((TN, C), jnp.float32),
    )(x2, y, num0, num1, den, dsel, te, mW1, mb1, mW2, mb2,
      Wo, bo, fW1, fb1, fW2, fb2)


def kernel(x, time_encoding, context_encoding, self_edges, cross_edges,
           mod_W1, mod_b1, mod_W2, mod_b2,
           siren_W1, siren_b1, siren_W2, siren_b2,
           Wq, bq, Wk, bk, Wv, bv, Wo, bo,
           ffn_W1, ffn_b1, ffn_W2, ffn_b2):
    x2 = x.reshape(TN, C)
    c7 = context_encoding.reshape(TN, 7)
    te = time_encoding.reshape(1, 256)

    hid = jnp.arange(C, dtype=jnp.int32) // D
    msel = (hid[:, None] == hid[None, :]).astype(jnp.float32)
    dsel = (jnp.arange(H, dtype=jnp.int32)[:, None] == hid[None, :]).astype(jnp.float32)

    s0, s1 = self_edges[:, 0], self_edges[:, 1]
    c0, c1 = cross_edges[:, 0], cross_edges[:, 1]
    pad = EPAD - E_SC
    zi = jnp.zeros((pad,), jnp.int32)
    dstg = jnp.stack([
        jnp.concatenate([s0, c0, zi]),
        jnp.concatenate([s0 + N, c0 + N, zi]),
    ]).reshape(K, TILES, NCHUNK, CHUNK)
    srcg = jnp.stack([
        jnp.concatenate([s1, c1 + N, zi]),
        jnp.concatenate([s1 + N, c1, zi]),
    ]).reshape(K, TILES, NCHUNK, CHUNK)
    dl = jnp.concatenate([s0, c0, jnp.full((pad,), N, jnp.int32)])
    dstl = jnp.stack([dl, dl]).reshape(K, TILES, NCHUNK, CHUNK)
    eidx = jnp.stack([dstg, srcg, dstl], axis=3)

    zrow = jnp.zeros((RPT, ACC_W), jnp.float32)

    b2 = lambda b: b.reshape(1, -1)
    y, q0, q1, kv0, kv1 = _tc_pre(x2, c7, te, msel,
                                  siren_W1, b2(siren_b1), siren_W2, b2(siren_b2),
                                  mod_W1, b2(mod_b1), mod_W2, b2(mod_b2),
                                  Wq, b2(bq), Wk, b2(bk), Wv, b2(bv))

    onum0, onum1, oden0, oden1 = _sc_edges(q0, q1, kv0, kv1, eidx, zrow)

    num0 = onum0[:, :N, :].reshape(TN, CG)
    num1 = onum1[:, :N, :].reshape(TN, CG)
    den = jnp.concatenate([oden0[:, :N, :HG], oden1[:, :N, :HG]],
                          axis=-1).reshape(TN, H)

    out = _tc_post(x2, y, num0, num1, den, dsel, te,
                   mod_W1, b2(mod_b1), mod_W2, b2(mod_b2),
                   Wo, b2(bo), ffn_W1, b2(ffn_b1), ffn_W2, b2(ffn_b2))
    return out.reshape(1, TN, C)


# speedup vs baseline: 5.0495x; 5.0495x over previous
import functools

import jax
import jax.numpy as jnp
from jax import lax
from jax.experimental import pallas as pl
from jax.experimental.pallas import tpu as pltpu
from jax.experimental.pallas import tpu_sc as plsc

N = 10000
K = 2
C = 128
H = 8
D = 16
ES = 160000
EC = 20000
TN = K * N

HG = H // 2
CG = HG * D
E_SC = ES + EC
TILES = 16
CHUNK = 128
NCHUNK = 88
EPT = NCHUNK * CHUNK
EPAD = TILES * EPT
ACC_ROWS = 10112
RPT = ACC_ROWS // TILES
ACC_W = CG + 16

ROWS_BLK = 800
GRID = TN // ROWS_BLK


def _silu(z):
    return z * jax.nn.sigmoid(z)


def _tc1_body(x_ref, c7_ref, te_ref, msel_ref,
              sW1_ref, sb1_ref, sW2_ref, sb2_ref,
              mW1_ref, mb1_ref, mW2_ref, mb2_ref,
              Wq_ref, bq_ref, Wk_ref, bk_ref, Wv_ref, bv_ref,
              y_ref, q0_ref, q1_ref, kv0_ref, kv1_ref):
    x = x_ref[...]
    ctx = jnp.sin(30.0 * (c7_ref[...] @ sW1_ref[...] + sb1_ref[...]))
    ctx = ctx @ sW2_ref[...] + sb2_ref[...]
    hmod = _silu(te_ref[...] @ mW1_ref[...] + mb1_ref[...])
    hmod = hmod @ mW2_ref[...] + mb2_ref[...]
    a = hmod[:, :C]
    bmod = hmod[:, C:2 * C]
    z = x + ctx
    mu = jnp.mean(z, axis=-1, keepdims=True)
    zc = z - mu
    var = jnp.mean(zc * zc, axis=-1, keepdims=True)
    y = (a + 1.0) * (zc * lax.rsqrt(var + 1e-5)) + bmod
    msel = msel_ref[...]
    q = y @ Wq_ref[...] + bq_ref[...]
    qn = q / (jnp.sqrt((q * q) @ msel) + 1e-6)
    k = y @ Wk_ref[...] + bk_ref[...]
    kn = k / (jnp.sqrt((k * k) @ msel) + 1e-6)
    v = y @ Wv_ref[...] + bv_ref[...]
    y_ref[...] = y
    q0_ref[...] = qn[:, :CG]
    q1_ref[...] = qn[:, CG:]
    kv0_ref[:, :CG] = kn[:, :CG]
    kv0_ref[:, CG:] = v[:, :CG]
    kv1_ref[:, :CG] = kn[:, CG:]
    kv1_ref[:, CG:] = v[:, CG:]


def _tc_pre(x2, c7, te, msel, sW1, sb1, sW2, sb2, mW1, mb1, mW2, mb2,
            Wq, bq, Wk, bk, Wv, bv):
    full = lambda s: pl.BlockSpec(s, lambda i: (0, 0))
    row = lambda w: pl.BlockSpec((ROWS_BLK, w), lambda i: (i, 0))
    return pl.pallas_call(
        _tc1_body,
        grid=(GRID,),
        in_specs=[row(C), row(7), full((1, 256)), full((C, C)),
                  full((7, C)), full((1, C)), full((C, C)), full((1, C)),
                  full((256, 256)), full((1, 256)), full((256, 3 * C)), full((1, 3 * C)),
                  full((C, C)), full((1, C)), full((C, C)), full((1, C)),
                  full((C, C)), full((1, C))],
        out_specs=[row(C), row(CG), row(CG), row(2 * CG), row(2 * CG)],
        out_shape=[jax.ShapeDtypeStruct((TN, C), jnp.float32),
                   jax.ShapeDtypeStruct((TN, CG), jnp.float32),
                   jax.ShapeDtypeStruct((TN, CG), jnp.float32),
                   jax.ShapeDtypeStruct((TN, 2 * CG), jnp.float32),
                   jax.ShapeDtypeStruct((TN, 2 * CG), jnp.float32)],
    )(x2, c7, te, msel, sW1, sb1, sW2, sb2, mW1, mb1, mW2, mb2,
      Wq, bq, Wk, bk, Wv, bv)


def _sc_edges(q0, q1, kv0, kv1, eidx, zrow):
    mesh = plsc.VectorSubcoreMesh(core_axis_name="c", subcore_axis_name="s")

    @functools.partial(
        pl.kernel,
        mesh=mesh,
        compiler_params=pltpu.CompilerParams(
            needs_layout_passes=False, use_tc_tiling_on_sc=False),
        out_type=[jax.ShapeDtypeStruct((K, ACC_ROWS, CG), jnp.float32),
                  jax.ShapeDtypeStruct((K, ACC_ROWS, CG), jnp.float32),
                  jax.ShapeDtypeStruct((K, ACC_ROWS, 16), jnp.float32),
                  jax.ShapeDtypeStruct((K, ACC_ROWS, 16), jnp.float32)],
        scratch_types=[
            pltpu.VMEM((4, 3, CHUNK), jnp.int32),
            pltpu.VMEM((2, CHUNK), jnp.int32),
            pltpu.VMEM((2, CHUNK, CG), jnp.float32),
            pltpu.VMEM((2, CHUNK, 2 * CG), jnp.float32),
            pltpu.VMEM((2, CHUNK, ACC_W), jnp.float32),
            pltpu.VMEM_SHARED((ACC_ROWS, ACC_W), jnp.float32),
            pltpu.SemaphoreType.DMA,
            pltpu.SemaphoreType.DMA,
            pltpu.SemaphoreType.DMA,
            pltpu.SemaphoreType.DMA,
            pltpu.SemaphoreType.DMA,
            pltpu.SemaphoreType.DMA,
            pltpu.SemaphoreType.DMA,
            pltpu.SemaphoreType.DMA,
        ],
    )
    def _k(q0_hbm, q1_hbm, kv0_hbm, kv1_hbm, eidx_hbm, zr_hbm,
           onum0_hbm, onum1_hbm, oden0_hbm, oden1_hbm,
           ibuf, sidx, qv, kvv, pbuf, acc,
           si0, si1, si2, si3, sg0, sg1, ss0, ss1):
        sem_i = (si0, si1, si2, si3)
        sem_g = (sg0, sg1)
        sem_s = (ss0, ss1)
        c = lax.axis_index("c")
        s = lax.axis_index("s")
        rows = pl.ds(s * RPT, RPT)
        pltpu.sync_copy(zr_hbm.at[pl.ds(0, CHUNK)], pbuf.at[0])
        pltpu.sync_copy(zr_hbm.at[pl.ds(0, CHUNK)], pbuf.at[1])
        lane = lax.iota(jnp.int32, 16)

        def idx_start(i, slot, c=c, s=s):
            pltpu.async_copy(eidx_hbm.at[c, s, i], ibuf.at[slot], sem_i[slot])

        def idx_wait(i, slot, c=c, s=s):
            pltpu.make_async_copy(
                eidx_hbm.at[c, s, i], ibuf.at[slot], sem_i[slot]).wait()

        for hg, (q_hbm, kv_hbm, onum_hbm, oden_hbm) in enumerate(
                [(q0_hbm, kv0_hbm, onum0_hbm, oden0_hbm),
                 (q1_hbm, kv1_hbm, onum1_hbm, oden1_hbm)]):
            pltpu.sync_copy(zr_hbm, acc.at[rows])
            plsc.subcore_barrier()

            def gathers_start(i, slot, g2, q_hbm=q_hbm, kv_hbm=kv_hbm):
                pltpu.async_copy(q_hbm.at[ibuf.at[slot, 0]], qv.at[g2],
                                 sem_g[g2])
                pltpu.async_copy(kv_hbm.at[ibuf.at[slot, 1]], kvv.at[g2],
                                 sem_g[g2])

            def gathers_wait(i, slot, g2, q_hbm=q_hbm, kv_hbm=kv_hbm):
                pltpu.make_async_copy(
                    q_hbm.at[ibuf.at[slot, 0]], qv.at[g2], sem_g[g2]).wait()
                pltpu.make_async_copy(
                    kv_hbm.at[ibuf.at[slot, 1]], kvv.at[g2], sem_g[g2]).wait()

            idx_start(0, 0)
            idx_start(1, 1)
            idx_wait(0, 0)
            gathers_start(0, 0, 0)
            idx_start(2, 2)

            def quad_body(ii, carry, hg=hg):
                for b in (0, 1, 2, 3):
                    i = 4 * ii + b
                    g2 = b % 2
                    gathers_wait(i, b, g2)

                    @pl.when(4 * ii + b >= 2)
                    def _():
                        pltpu.make_async_copy(
                            pbuf.at[g2], acc.at[sidx.at[g2]], sem_s[g2]).wait()

                    @pl.when(i + 3 < NCHUNK)
                    def _():
                        idx_start(i + 3, (b + 3) % 4)

                    @pl.when(i + 1 < NCHUNK)
                    def _():
                        idx_wait(i + 1, (b + 1) % 4)
                        gathers_start(i + 1, (b + 1) % 4, (g2 + 1) % 2)

                    @plsc.parallel_loop(0, CHUNK, unroll=4)
                    def _edge_body(e, g2=g2):
                        pvec = jnp.zeros((16,), jnp.float32)
                        for h in range(HG):
                            qh = qv[g2, e, pl.ds(h * D, D)]
                            kh = kvv[g2, e, pl.ds(h * D, D)]
                            dot = jnp.sum(qh * kh)
                            pexp = jnp.exp(jnp.full((16,), dot * 0.25,
                                                    jnp.float32))
                            pvec = jnp.where(lane == h, pexp, pvec)
                            vh = kvv[g2, e, pl.ds(CG + h * D, D)]
                            pbuf[g2, e, pl.ds(h * D, D)] = pexp * vh
                        pbuf[g2, e, pl.ds(CG, 16)] = pvec
                    for t in range(CHUNK // 16):
                        sidx[g2, pl.ds(t * 16, 16)] = ibuf[b, 2, pl.ds(t * 16, 16)]
                    pltpu.async_copy(pbuf.at[g2], acc.at[sidx.at[g2]],
                                     sem_s[g2], add=True)
                return carry

            lax.fori_loop(0, NCHUNK // 4, quad_body, 0)
            for g2 in (0, 1):
                pltpu.make_async_copy(
                    pbuf.at[g2], acc.at[sidx.at[g2]], sem_s[g2]).wait()
            plsc.subcore_barrier()
            pltpu.sync_copy(acc.at[rows, pl.ds(0, CG)], onum_hbm.at[c, rows])
            pltpu.sync_copy(acc.at[rows, pl.ds(CG, 16)], oden_hbm.at[c, rows])

    return _k(q0, q1, kv0, kv1, eidx, zrow)


def _tc2_body(x_ref, y_ref, num0_ref, num1_ref, den_ref, dsel_ref, te_ref,
              mW1_ref, mb1_ref, mW2_ref, mb2_ref,
              Wo_ref, bo_ref, fW1_ref, fb1_ref, fW2_ref, fb2_ref,
              o_ref):
    hmod = _silu(te_ref[...] @ mW1_ref[...] + mb1_ref[...])
    hmod = hmod @ mW2_ref[...] + mb2_ref[...]
    cmod = hmod[:, 2 * C:]
    rden = 1.0 / (den_ref[...] @ dsel_ref[...] + 1e-9)
    attn = ((num0_ref[...] * rden[:, :CG]) @ Wo_ref[:CG, :]
            + (num1_ref[...] * rden[:, CG:]) @ Wo_ref[CG:, :] + bo_ref[...])
    y2 = y_ref[...] + attn
    f = _silu(y2 @ fW1_ref[...] + fb1_ref[...]) @ fW2_ref[...] + fb2_ref[...]
    o_ref[...] = (x_ref[...] + cmod * f) * lax.rsqrt(1.0 + cmod * cmod)


def _tc_post(x2, y, num0, num1, den, dsel, te, mW1, mb1, mW2, mb2,
             Wo, bo, fW1, fb1, fW2, fb2):
    full = lambda s: pl.BlockSpec(s, lambda i: (0, 0))
    row = lambda w: pl.BlockSpec((ROWS_BLK, w), lambda i: (i, 0))
    return pl.pallas_call(
        _tc2_body,
        grid=(GRID,),
        in_specs=[row(C), row(C), row(CG), row(CG), row(H), full((H, C)),
                  full((1, 256)),
                  full((256, 256)), full((1, 256)), full((256, 3 * C)), full((1, 3 * C)),
                  full((C, C)), full((1, C)),
                  full((C, 4 * C)), full((1, 4 * C)), full((4 * C, C)), full((1, C))],
        out_specs=row(C),
        out_shape=jax.ShapeDtypeStruct((TN, C), jnp.float32),
    )(x2, y, num0, num1, den, dsel, te, mW1, mb1, mW2, mb2,
      Wo, bo, fW1, fb1, fW2, fb2)


def kernel(x, time_encoding, context_encoding, self_edges, cross_edges,
           mod_W1, mod_b1, mod_W2, mod_b2,
           siren_W1, siren_b1, siren_W2, siren_b2,
           Wq, bq, Wk, bk, Wv, bv, Wo, bo,
           ffn_W1, ffn_b1, ffn_W2, ffn_b2):
    x2 = x.reshape(TN, C)
    c7 = context_encoding.reshape(TN, 7)
    te = time_encoding.reshape(1, 256)

    hid = jnp.arange(C, dtype=jnp.int32) // D
    msel = (hid[:, None] == hid[None, :]).astype(jnp.float32)
    dsel = (jnp.arange(H, dtype=jnp.int32)[:, None] == hid[None, :]).astype(jnp.float32)

    s0, s1 = self_edges[:, 0], self_edges[:, 1]
    c0, c1 = cross_edges[:, 0], cross_edges[:, 1]
    pad = EPAD - E_SC
    zi = jnp.zeros((pad,), jnp.int32)
    dstg = jnp.stack([
        jnp.concatenate([s0, c0, zi]),
        jnp.concatenate([s0 + N, c0 + N, zi]),
    ]).reshape(K, TILES, NCHUNK, CHUNK)
    srcg = jnp.stack([
        jnp.concatenate([s1, c1 + N, zi]),
        jnp.concatenate([s1 + N, c1, zi]),
    ]).reshape(K, TILES, NCHUNK, CHUNK)
    dl = jnp.concatenate([s0, c0, jnp.full((pad,), N, jnp.int32)])
    dstl = jnp.stack([dl, dl]).reshape(K, TILES, NCHUNK, CHUNK)
    eidx = jnp.stack([dstg, srcg, dstl], axis=3)

    zrow = jnp.zeros((RPT, ACC_W), jnp.float32)

    b2 = lambda b: b.reshape(1, -1)
    y, q0, q1, kv0, kv1 = _tc_pre(x2, c7, te, msel,
                                  siren_W1, b2(siren_b1), siren_W2, b2(siren_b2),
                                  mod_W1, b2(mod_b1), mod_W2, b2(mod_b2),
                                  Wq, b2(bq), Wk, b2(bk), Wv, b2(bv))

    onum0, onum1, oden0, oden1 = _sc_edges(q0, q1, kv0, kv1, eidx, zrow)

    num0 = onum0[:, :N, :].reshape(TN, CG)
    num1 = onum1[:, :N, :].reshape(TN, CG)
    den = jnp.concatenate([oden0[:, :N, :HG], oden1[:, :N, :HG]],
                          axis=-1).reshape(TN, H)

    out = _tc_post(x2, y, num0, num1, den, dsel, te,
                   mod_W1, b2(mod_b1), mod_W2, b2(mod_b2),
                   Wo, b2(bo), ffn_W1, b2(ffn_b1), ffn_W2, b2(ffn_b2))
    return out.reshape(1, TN, C)


# speedup vs baseline: 5.3685x; 1.0632x over previous
import functools

import jax
import jax.numpy as jnp
from jax import lax
from jax.experimental import pallas as pl
from jax.experimental.pallas import tpu as pltpu
from jax.experimental.pallas import tpu_sc as plsc

N = 10000
K = 2
C = 128
H = 8
D = 16
ES = 160000
EC = 20000
TN = K * N

HG = H // 2
CG = HG * D
E_SC = ES + EC
TILES = 16
CHUNK = 128
NCHUNK = 88
EPT = NCHUNK * CHUNK
EPAD = TILES * EPT
ACC_ROWS = 10112
RPT = ACC_ROWS // TILES
ACC_W = CG + 16

ROWS_BLK = 800
GRID = TN // ROWS_BLK


def _silu(z):
    return z * jax.nn.sigmoid(z)


def _tc1_body(x_ref, c7_ref, te_ref, msel_ref, P_ref,
              sW1_ref, sb1_ref, sW2_ref, sb2_ref,
              mW1_ref, mb1_ref, mW2_ref, mb2_ref,
              Wq_ref, bq_ref, Wk_ref, bk_ref, Wv_ref, bv_ref,
              y_ref, q0_ref, q1_ref, kv0_ref, kv1_ref):
    x = x_ref[...]
    ctx = jnp.sin(30.0 * (c7_ref[...] @ sW1_ref[...] + sb1_ref[...]))
    ctx = ctx @ sW2_ref[...] + sb2_ref[...]
    hmod = _silu(te_ref[...] @ mW1_ref[...] + mb1_ref[...])
    hmod = hmod @ mW2_ref[...] + mb2_ref[...]
    a = hmod[:, :C]
    bmod = hmod[:, C:2 * C]
    z = x + ctx
    mu = jnp.mean(z, axis=-1, keepdims=True)
    zc = z - mu
    var = jnp.mean(zc * zc, axis=-1, keepdims=True)
    y = (a + 1.0) * (zc * lax.rsqrt(var + 1e-5)) + bmod
    msel = msel_ref[...]
    q = y @ Wq_ref[...] + bq_ref[...]
    qn = q / (jnp.sqrt((q * q) @ msel) + 1e-6)
    k = y @ Wk_ref[...] + bk_ref[...]
    kn = k / (jnp.sqrt((k * k) @ msel) + 1e-6)
    v = y @ Wv_ref[...] + bv_ref[...]
    y_ref[...] = y
    P = P_ref[...]
    bf = jnp.bfloat16
    q0_ref[...] = (qn[:, :CG] @ P).astype(bf)
    q1_ref[...] = (qn[:, CG:] @ P).astype(bf)
    kv0_ref[:, :CG] = (kn[:, :CG] @ P).astype(bf)
    kv0_ref[:, CG:] = (v[:, :CG] @ P).astype(bf)
    kv1_ref[:, :CG] = (kn[:, CG:] @ P).astype(bf)
    kv1_ref[:, CG:] = (v[:, CG:] @ P).astype(bf)


def _tc_pre(x2, c7, te, msel, P, sW1, sb1, sW2, sb2, mW1, mb1, mW2, mb2,
            Wq, bq, Wk, bk, Wv, bv):
    full = lambda s: pl.BlockSpec(s, lambda i: (0, 0))
    row = lambda w: pl.BlockSpec((ROWS_BLK, w), lambda i: (i, 0))
    return pl.pallas_call(
        _tc1_body,
        grid=(GRID,),
        in_specs=[row(C), row(7), full((1, 256)), full((C, C)), full((CG, CG)),
                  full((7, C)), full((1, C)), full((C, C)), full((1, C)),
                  full((256, 256)), full((1, 256)), full((256, 3 * C)), full((1, 3 * C)),
                  full((C, C)), full((1, C)), full((C, C)), full((1, C)),
                  full((C, C)), full((1, C))],
        out_specs=[row(C), row(CG), row(CG), row(2 * CG), row(2 * CG)],
        out_shape=[jax.ShapeDtypeStruct((TN, C), jnp.float32),
                   jax.ShapeDtypeStruct((TN, CG), jnp.bfloat16),
                   jax.ShapeDtypeStruct((TN, CG), jnp.bfloat16),
                   jax.ShapeDtypeStruct((TN, 2 * CG), jnp.bfloat16),
                   jax.ShapeDtypeStruct((TN, 2 * CG), jnp.bfloat16)],
    )(x2, c7, te, msel, P, sW1, sb1, sW2, sb2, mW1, mb1, mW2, mb2,
      Wq, bq, Wk, bk, Wv, bv)


def _sc_edges(q0, q1, kv0, kv1, eidx, zrow):
    mesh = plsc.VectorSubcoreMesh(core_axis_name="c", subcore_axis_name="s")

    @functools.partial(
        pl.kernel,
        mesh=mesh,
        compiler_params=pltpu.CompilerParams(
            needs_layout_passes=False, use_tc_tiling_on_sc=False),
        out_type=[jax.ShapeDtypeStruct((K, ACC_ROWS, CG), jnp.float32),
                  jax.ShapeDtypeStruct((K, ACC_ROWS, CG), jnp.float32),
                  jax.ShapeDtypeStruct((K, ACC_ROWS, 16), jnp.float32),
                  jax.ShapeDtypeStruct((K, ACC_ROWS, 16), jnp.float32)],
        scratch_types=[
            pltpu.VMEM((4, 3, CHUNK), jnp.int32),
            pltpu.VMEM((2, CHUNK), jnp.int32),
            pltpu.VMEM((2, CHUNK, CG), jnp.bfloat16),
            pltpu.VMEM((2, CHUNK, 2 * CG), jnp.bfloat16),
            pltpu.VMEM((2, CHUNK, ACC_W), jnp.float32),
            pltpu.VMEM_SHARED((ACC_ROWS, ACC_W), jnp.float32),
            pltpu.SemaphoreType.DMA,
            pltpu.SemaphoreType.DMA,
            pltpu.SemaphoreType.DMA,
            pltpu.SemaphoreType.DMA,
            pltpu.SemaphoreType.DMA,
            pltpu.SemaphoreType.DMA,
            pltpu.SemaphoreType.DMA,
            pltpu.SemaphoreType.DMA,
        ],
    )
    def _k(q0_hbm, q1_hbm, kv0_hbm, kv1_hbm, eidx_hbm, zr_hbm,
           onum0_hbm, onum1_hbm, oden0_hbm, oden1_hbm,
           ibuf, sidx, qv, kvv, pbuf, acc,
           si0, si1, si2, si3, sg0, sg1, ss0, ss1):
        sem_i = (si0, si1, si2, si3)
        sem_g = (sg0, sg1)
        sem_s = (ss0, ss1)
        c = lax.axis_index("c")
        s = lax.axis_index("s")
        rows = pl.ds(s * RPT, RPT)
        pltpu.sync_copy(zr_hbm.at[pl.ds(0, CHUNK)], pbuf.at[0])
        pltpu.sync_copy(zr_hbm.at[pl.ds(0, CHUNK)], pbuf.at[1])
        lane = lax.iota(jnp.int32, 16)

        def idx_start(i, slot, c=c, s=s):
            pltpu.async_copy(eidx_hbm.at[c, s, i], ibuf.at[slot], sem_i[slot])

        def idx_wait(i, slot, c=c, s=s):
            pltpu.make_async_copy(
                eidx_hbm.at[c, s, i], ibuf.at[slot], sem_i[slot]).wait()

        for hg, (q_hbm, kv_hbm, onum_hbm, oden_hbm) in enumerate(
                [(q0_hbm, kv0_hbm, onum0_hbm, oden0_hbm),
                 (q1_hbm, kv1_hbm, onum1_hbm, oden1_hbm)]):
            pltpu.sync_copy(zr_hbm, acc.at[rows])
            plsc.subcore_barrier()

            def gathers_start(i, slot, g2, q_hbm=q_hbm, kv_hbm=kv_hbm):
                pltpu.async_copy(q_hbm.at[ibuf.at[slot, 0]], qv.at[g2],
                                 sem_g[g2])
                pltpu.async_copy(kv_hbm.at[ibuf.at[slot, 1]], kvv.at[g2],
                                 sem_g[g2])

            def gathers_wait(i, slot, g2, q_hbm=q_hbm, kv_hbm=kv_hbm):
                pltpu.make_async_copy(
                    q_hbm.at[ibuf.at[slot, 0]], qv.at[g2], sem_g[g2]).wait()
                pltpu.make_async_copy(
                    kv_hbm.at[ibuf.at[slot, 1]], kvv.at[g2], sem_g[g2]).wait()

            idx_start(0, 0)
            idx_start(1, 1)
            idx_wait(0, 0)
            gathers_start(0, 0, 0)
            idx_start(2, 2)

            def quad_body(ii, carry, hg=hg):
                for b in (0, 1, 2, 3):
                    i = 4 * ii + b
                    g2 = b % 2
                    gathers_wait(i, b, g2)

                    @pl.when(4 * ii + b >= 2)
                    def _():
                        pltpu.make_async_copy(
                            pbuf.at[g2], acc.at[sidx.at[g2]], sem_s[g2]).wait()

                    @pl.when(i + 3 < NCHUNK)
                    def _():
                        idx_start(i + 3, (b + 3) % 4)

                    @pl.when(i + 1 < NCHUNK)
                    def _():
                        idx_wait(i + 1, (b + 1) % 4)
                        gathers_start(i + 1, (b + 1) % 4, (g2 + 1) % 2)

                    @plsc.parallel_loop(0, CHUNK, unroll=4)
                    def _edge_body(e, g2=g2):
                        pvec = jnp.zeros((16,), jnp.float32)
                        for t in range(HG // 2):
                            qb = qv[g2, e, pl.ds(2 * D * t, 2 * D)]
                            kb = kvv[g2, e, pl.ds(2 * D * t, 2 * D)]
                            vb = kvv[g2, e, pl.ds(CG + 2 * D * t, 2 * D)]
                            qu = plsc.unpack(qb, format=plsc.PackFormat.INTERLEAVED)
                            ku = plsc.unpack(kb, format=plsc.PackFormat.INTERLEAVED)
                            vu = plsc.unpack(vb, format=plsc.PackFormat.INTERLEAVED)
                            for r in range(2):
                                h = 2 * t + r
                                dot = jnp.sum(qu[r] * ku[r])
                                pexp = jnp.exp(jnp.full((16,), dot * 0.25,
                                                        jnp.float32))
                                pvec = jnp.where(lane == h, pexp, pvec)
                                pbuf[g2, e, pl.ds(h * D, D)] = pexp * vu[r]
                        pbuf[g2, e, pl.ds(CG, 16)] = pvec
                    for t in range(CHUNK // 16):
                        sidx[g2, pl.ds(t * 16, 16)] = ibuf[b, 2, pl.ds(t * 16, 16)]
                    pltpu.async_copy(pbuf.at[g2], acc.at[sidx.at[g2]],
                                     sem_s[g2], add=True)
                return carry

            lax.fori_loop(0, NCHUNK // 4, quad_body, 0)
            for g2 in (0, 1):
                pltpu.make_async_copy(
                    pbuf.at[g2], acc.at[sidx.at[g2]], sem_s[g2]).wait()
            plsc.subcore_barrier()
            pltpu.sync_copy(acc.at[rows, pl.ds(0, CG)], onum_hbm.at[c, rows])
            pltpu.sync_copy(acc.at[rows, pl.ds(CG, 16)], oden_hbm.at[c, rows])

    return _k(q0, q1, kv0, kv1, eidx, zrow)


def _tc2_body(x_ref, y_ref, num0_ref, num1_ref, den_ref, dsel_ref, te_ref,
              mW1_ref, mb1_ref, mW2_ref, mb2_ref,
              Wo_ref, bo_ref, fW1_ref, fb1_ref, fW2_ref, fb2_ref,
              o_ref):
    hmod = _silu(te_ref[...] @ mW1_ref[...] + mb1_ref[...])
    hmod = hmod @ mW2_ref[...] + mb2_ref[...]
    cmod = hmod[:, 2 * C:]
    rden = 1.0 / (den_ref[...] @ dsel_ref[...] + 1e-9)
    attn = ((num0_ref[...] * rden[:, :CG]) @ Wo_ref[:CG, :]
            + (num1_ref[...] * rden[:, CG:]) @ Wo_ref[CG:, :] + bo_ref[...])
    y2 = y_ref[...] + attn
    f = _silu(y2 @ fW1_ref[...] + fb1_ref[...]) @ fW2_ref[...] + fb2_ref[...]
    o_ref[...] = (x_ref[...] + cmod * f) * lax.rsqrt(1.0 + cmod * cmod)


def _tc_post(x2, y, num0, num1, den, dsel, te, mW1, mb1, mW2, mb2,
             Wo, bo, fW1, fb1, fW2, fb2):
    full = lambda s: pl.BlockSpec(s, lambda i: (0, 0))
    row = lambda w: pl.BlockSpec((ROWS_BLK, w), lambda i: (i, 0))
    return pl.pallas_call(
        _tc2_body,
        grid=(GRID,),
        in_specs=[row(C), row(C), row(CG), row(CG), row(H), full((H, C)),
                  full((1, 256)),
                  full((256, 256)), full((1, 256)), full((256, 3 * C)), full((1, 3 * C)),
                  full((C, C)), full((1, C)),
                  full((C, 4 * C)), full((1, 4 * C)), full((4 * C, C)), full((1, C))],
        out_specs=row(C),
        out_shape=jax.ShapeDtypeStruct((TN, C), jnp.float32),
    )(x2, y, num0, num1, den, dsel, te, mW1, mb1, mW2, mb2,
      Wo, bo, fW1, fb1, fW2, fb2)


def kernel(x, time_encoding, context_encoding, self_edges, cross_edges,
           mod_W1, mod_b1, mod_W2, mod_b2,
           siren_W1, siren_b1, siren_W2, siren_b2,
           Wq, bq, Wk, bk, Wv, bv, Wo, bo,
           ffn_W1, ffn_b1, ffn_W2, ffn_b2):
    x2 = x.reshape(TN, C)
    c7 = context_encoding.reshape(TN, 7)
    te = time_encoding.reshape(1, 256)

    hid = jnp.arange(C, dtype=jnp.int32) // D
    msel = (hid[:, None] == hid[None, :]).astype(jnp.float32)
    dsel = (jnp.arange(H, dtype=jnp.int32)[:, None] == hid[None, :]).astype(jnp.float32)
    i64 = jnp.arange(CG, dtype=jnp.int32)
    t64, rem = i64 // 32, i64 % 32
    dst64 = t64 * 32 + 2 * (rem % D) + rem // D
    P = (dst64[:, None] == i64[None, :]).astype(jnp.float32)

    s0, s1 = self_edges[:, 0], self_edges[:, 1]
    c0, c1 = cross_edges[:, 0], cross_edges[:, 1]
    pad = EPAD - E_SC
    zi = jnp.zeros((pad,), jnp.int32)
    dstg = jnp.stack([
        jnp.concatenate([s0, c0, zi]),
        jnp.concatenate([s0 + N, c0 + N, zi]),
    ]).reshape(K, TILES, NCHUNK, CHUNK)
    srcg = jnp.stack([
        jnp.concatenate([s1, c1 + N, zi]),
        jnp.concatenate([s1 + N, c1, zi]),
    ]).reshape(K, TILES, NCHUNK, CHUNK)
    dl = jnp.concatenate([s0, c0, jnp.full((pad,), N, jnp.int32)])
    dstl = jnp.stack([dl, dl]).reshape(K, TILES, NCHUNK, CHUNK)
    eidx = jnp.stack([dstg, srcg, dstl], axis=3)

    zrow = jnp.zeros((RPT, ACC_W), jnp.float32)

    b2 = lambda b: b.reshape(1, -1)
    y, q0, q1, kv0, kv1 = _tc_pre(x2, c7, te, msel, P,
                                  siren_W1, b2(siren_b1), siren_W2, b2(siren_b2),
                                  mod_W1, b2(mod_b1), mod_W2, b2(mod_b2),
                                  Wq, b2(bq), Wk, b2(bk), Wv, b2(bv))

    onum0, onum1, oden0, oden1 = _sc_edges(q0, q1, kv0, kv1, eidx, zrow)

    num0 = onum0[:, :N, :].reshape(TN, CG)
    num1 = onum1[:, :N, :].reshape(TN, CG)
    den = jnp.concatenate([oden0[:, :N, :HG], oden1[:, :N, :HG]],
                          axis=-1).reshape(TN, H)

    out = _tc_post(x2, y, num0, num1, den, dsel, te,
                   mod_W1, b2(mod_b1), mod_W2, b2(mod_b2),
                   Wo, b2(bo), ffn_W1, b2(ffn_b1), ffn_W2, b2(ffn_b2))
    return out.reshape(1, TN, C)
